# baseline (speedup 1.0000x reference)
import functools
import math

import numpy as np
import jax
import jax.numpy as jnp
from jax import lax
from jax.experimental import pallas as pl
from jax.experimental.pallas import tpu as pltpu
from jax.experimental.pallas import tpu_sc as plsc

_FACTOR = 1


@functools.lru_cache(maxsize=None)
def _dft_mats(L: int, FP: int):
    F = L // 2 + 1
    f = np.arange(FP, dtype=np.int64)[:, None]
    t = np.arange(L, dtype=np.int64)[None, :]
    ang = 2.0 * np.pi * ((f * t) % L).astype(np.float64) / L
    valid = (f < F).astype(np.float64)
    cr = (np.cos(ang) * valid).astype(np.float32)
    ci = (-np.sin(ang) * valid).astype(np.float32)
    alpha = np.where((f == 0) | (f == L // 2), 1.0, 2.0) * valid
    dr = ((np.cos(ang) * alpha / L).T).astype(np.float32)
    di = ((-np.sin(ang) * alpha / L).T).astype(np.float32)
    return cr, ci, dr, di


def _split_bf16(x):
    hi = x.astype(jnp.bfloat16)
    lo = (x - hi.astype(jnp.float32)).astype(jnp.bfloat16)
    return hi, lo


def _mm3(ah, al, bh, bl):
    f32 = jnp.float32
    return (jnp.dot(ah, bh, preferred_element_type=f32)
            + jnp.dot(ah, bl, preferred_element_type=f32)
            + jnp.dot(al, bh, preferred_element_type=f32))


def _mm3t(ah, al, bh, bl):
    f32 = jnp.float32
    dn = (((0,), (0,)), ((), ()))
    return (lax.dot_general(ah, bh, dn, preferred_element_type=f32)
            + lax.dot_general(ah, bl, dn, preferred_element_type=f32)
            + lax.dot_general(al, bh, dn, preferred_element_type=f32))


def _corr_topk_body(top_k, L, FP, CB, qth, kth,
                    mh0, mh1, mh2, mh3,
                    w_out, d_out,
                    s0, s1, s2, s3, sem):
    f32 = jnp.float32
    hbm_mats = [mh0, mh1, mh2, mh3]
    scr_mats = [s0, s1, s2, s3]

    @pl.when(pl.program_id(0) == 0)
    def _():
        cps = [pltpu.make_async_copy(src, dst, sem)
               for src, dst in zip(hbm_mats, scr_mats)]
        for cp in cps:
            cp.start()
        for cp in cps:
            cp.wait()

    crh, crl, cih, cil = (s[...] for s in scr_mats)
    qh, ql = _split_bf16(qth[...])
    kh, kl = _split_bf16(kth[...])
    qr = _mm3(crh, crl, qh, ql)
    qi = _mm3(cih, cil, qh, ql)
    kr = _mm3(crh, crl, kh, kl)
    ki = _mm3(cih, cil, kh, kl)
    fidx = lax.broadcasted_iota(jnp.int32, (FP, CB), 0)
    a = jnp.where((fidx == 0) | (fidx == L // 2), 1.0, 2.0).astype(f32)
    a = jnp.where(fidx <= L // 2, a, 0.0) * f32(1.0 / L)
    rr = (qr * kr + qi * ki) * a
    ri = (qi * kr - qr * ki) * a
    rrh, rrl = _split_bf16(rr)
    rih, ril = _split_bf16(ri)
    c = _mm3t(crh, crl, rrh, rrl) + _mm3t(cih, cil, rih, ril)

    iot = lax.broadcasted_iota(jnp.int32, (L, CB), 0)
    ws, ds = [], []
    for _i in range(top_k):
        mx = jnp.max(c, axis=0, keepdims=True)
        eq = c >= mx
        idx = jnp.min(jnp.where(eq, iot, L), axis=0, keepdims=True)
        ws.append(mx)
        ds.append(idx)
        c = jnp.where(iot == idx, f32(-3.0e38), c)
    w = jnp.concatenate(ws, axis=0)
    d = jnp.concatenate(ds, axis=0)
    m = jnp.max(w, axis=0, keepdims=True)
    e = jnp.exp(w - m)
    w = e / jnp.sum(e, axis=0, keepdims=True)
    pad = 16 - top_k
    w_out[...] = jnp.concatenate([w, jnp.zeros((pad, CB), f32)], axis=0)
    d_out[...] = jnp.concatenate(
        [d, jnp.zeros((pad, CB), jnp.int32)], axis=0)


@functools.lru_cache(maxsize=None)
def _dft_mats_split(L: int, FP: int):
    import ml_dtypes
    out = []
    for m in _dft_mats(L, FP)[:2]:
        hi = m.astype(ml_dtypes.bfloat16)
        lo = (m - hi.astype(np.float32)).astype(ml_dtypes.bfloat16)
        out.append(hi)
        out.append(lo)
    return tuple(out)


def _corr_topk(qth, kth, top_k, CB=256, FP=1152, interpret=False):
    L, C = qth.shape
    mats = _dft_mats_split(L, FP)
    body = functools.partial(_corr_topk_body, top_k, L, FP, CB)
    grid = (C // CB,)
    bf16 = jnp.bfloat16
    w16, d16 = pl.pallas_call(
        body,
        grid=grid,
        in_specs=[pl.BlockSpec((L, CB), lambda j: (0, j))] * 2
        + [pl.BlockSpec(memory_space=pltpu.MemorySpace.HBM)] * 4,
        out_specs=[
            pl.BlockSpec((16, CB), lambda j: (0, j)),
            pl.BlockSpec((16, CB), lambda j: (0, j)),
        ],
        out_shape=[
            jax.ShapeDtypeStruct((16, C), jnp.float32),
            jax.ShapeDtypeStruct((16, C), jnp.int32),
        ],
        scratch_shapes=[pltpu.VMEM((FP, L), bf16)] * 4
        + [pltpu.SemaphoreType.DMA],
        compiler_params=pltpu.CompilerParams(
            vmem_limit_bytes=63 * 1024 * 1024),
        interpret=interpret,
    )(qth, kth, *[jnp.asarray(m) for m in mats])
    return w16, d16


def _sc_agg(vt, wt, dt, top_k):
    C, L = vt.shape
    info = plsc.get_sparse_core_info()
    nw = info.num_cores * info.num_subcores
    cols_per = C // nw
    mesh = plsc.VectorSubcoreMesh(core_axis_name="c", subcore_axis_name="s")

    NBUF = 2

    @functools.partial(
        pl.kernel,
        out_type=jax.ShapeDtypeStruct((C, L), jnp.float32),
        mesh=mesh,
        scratch_types=[
            pltpu.VMEM((2 * L,), jnp.float32),
            pltpu.VMEM((2 * L,), jnp.float32),
            pltpu.VMEM((L,), jnp.float32),
            pltpu.VMEM((L,), jnp.float32),
            pltpu.VMEM((cols_per, 16), jnp.float32),
            pltpu.VMEM((cols_per, 16), jnp.int32),
            pltpu.SemaphoreType.DMA,
            pltpu.SemaphoreType.DMA,
            pltpu.SemaphoreType.DMA,
            pltpu.SemaphoreType.DMA,
        ],
    )
    def body(vt_hbm, wt_hbm, dt_hbm, out_hbm, vb0, vb1, ob0, ob1,
             wall, dall, si0, si1, so0, so1):
        vbufs = [vb0, vb1]
        obufs = [ob0, ob1]
        sin = [si0, si1]
        sout = [so0, so1]
        wid = lax.axis_index("s") * info.num_cores + lax.axis_index("c")
        base_col = wid * cols_per
        pltpu.sync_copy(wt_hbm.at[pl.ds(base_col, cols_per)], wall)
        pltpu.sync_copy(dt_hbm.at[pl.ds(base_col, cols_per)], dall)

        def in_copies(c0, b):
            return (
                pltpu.make_async_copy(
                    vt_hbm.at[c0], vbufs[b].at[pl.ds(0, L)], sin[b]),
                pltpu.make_async_copy(
                    vt_hbm.at[c0], vbufs[b].at[pl.ds(L, L)], sin[b]),
            )

        for b in range(NBUF):
            for cp in in_copies(base_col + b, b):
                cp.start()

        def outer(g, carry):
            for b in range(NBUF):
                j = g * NBUF + b
                c0 = base_col + j
                for cp in in_copies(c0, b):
                    cp.wait()

                @pl.when(g > 0)
                def _():
                    pltpu.make_async_copy(
                        obufs[b], out_hbm.at[c0 - NBUF], sout[b]).wait()

                wv = wall[j]
                dv = dall[j]
                wss = [wv[i] for i in range(top_k)]
                dss = [dv[i] for i in range(top_k)]
                vb = vbufs[b]
                ob = obufs[b]

                def vec_body(v, carry2):
                    base = v * 16
                    acc = wss[0] * vb[pl.ds(base + dss[0], 16)]
                    for i in range(1, top_k):
                        acc = acc + wss[i] * vb[pl.ds(base + dss[i], 16)]
                    ob[pl.ds(base, 16)] = acc
                    return carry2

                lax.fori_loop(0, L // 16, vec_body, 0, unroll=2)
                pltpu.async_copy(ob, out_hbm.at[c0], sout[b])

                @pl.when(j + NBUF < cols_per)
                def _():
                    for cp in in_copies(c0 + NBUF, b):
                        cp.start()

            return carry

        lax.fori_loop(0, cols_per // NBUF, outer, 0)
        for b in range(NBUF):
            pltpu.make_async_copy(
                obufs[b], out_hbm.at[base_col + cols_per - NBUF + b],
                sout[b]).wait()

    return body(vt, wt, dt)


def kernel(Q, K, V):
    B, H, L, D = Q.shape
    C = B * H * D
    top_k = int(_FACTOR * math.log(L))
    nh = 2
    bh = B // nh
    ch = bh * H * D

    outs = []
    for h in range(nh):
        qs = Q[h * bh:(h + 1) * bh]
        ks = K[h * bh:(h + 1) * bh]
        vs = V[h * bh:(h + 1) * bh]
        qt = jnp.transpose(qs, (2, 0, 1, 3)).reshape(L, ch)
        kt = jnp.transpose(ks, (2, 0, 1, 3)).reshape(L, ch)
        w16, d16 = _corr_topk(qt, kt, top_k)

        vt = jnp.transpose(vs, (0, 1, 3, 2)).reshape(ch, L)
        wt = jnp.transpose(w16, (1, 0))
        dt = jnp.transpose(d16, (1, 0))
        out_t = _sc_agg(vt, wt, dt, top_k)
        outs.append(
            jnp.transpose(out_t.reshape(bh, H, D, L), (0, 1, 3, 2)))
    return jnp.concatenate(outs, axis=0)


# speedup vs baseline: 1.0305x; 1.0305x over previous
import functools
import math

import numpy as np
import jax
import jax.numpy as jnp
from jax import lax
from jax.experimental import pallas as pl
from jax.experimental.pallas import tpu as pltpu
from jax.experimental.pallas import tpu_sc as plsc

_FACTOR = 1


@functools.lru_cache(maxsize=None)
def _dft_mats(L: int, FP: int):
    F = L // 2 + 1
    f = np.arange(FP, dtype=np.int64)[:, None]
    t = np.arange(L, dtype=np.int64)[None, :]
    ang = 2.0 * np.pi * ((f * t) % L).astype(np.float64) / L
    valid = (f < F).astype(np.float64)
    cr = (np.cos(ang) * valid).astype(np.float32)
    ci = (-np.sin(ang) * valid).astype(np.float32)
    alpha = np.where((f == 0) | (f == L // 2), 1.0, 2.0) * valid
    dr = ((np.cos(ang) * alpha / L).T).astype(np.float32)
    di = ((-np.sin(ang) * alpha / L).T).astype(np.float32)
    return cr, ci, dr, di


def _split_bf16(x):
    hi = x.astype(jnp.bfloat16)
    lo = (x - hi.astype(jnp.float32)).astype(jnp.bfloat16)
    return hi, lo


def _mm3(ah, al, bh, bl):
    f32 = jnp.float32
    return (jnp.dot(ah, bh, preferred_element_type=f32)
            + jnp.dot(ah, bl, preferred_element_type=f32)
            + jnp.dot(al, bh, preferred_element_type=f32))


def _mm3t(ah, al, bh, bl):
    f32 = jnp.float32
    dn = (((0,), (0,)), ((), ()))
    return (lax.dot_general(ah, bh, dn, preferred_element_type=f32)
            + lax.dot_general(ah, bl, dn, preferred_element_type=f32)
            + lax.dot_general(al, bh, dn, preferred_element_type=f32))


def _corr_topk_body(top_k, L, FP, CB, qth, kth,
                    mh0, mh1, mh2, mh3,
                    w_out, d_out,
                    s0, s1, s2, s3, sem):
    f32 = jnp.float32
    hbm_mats = [mh0, mh1, mh2, mh3]
    scr_mats = [s0, s1, s2, s3]

    @pl.when(pl.program_id(0) == 0)
    def _():
        cps = [pltpu.make_async_copy(src, dst, sem)
               for src, dst in zip(hbm_mats, scr_mats)]
        for cp in cps:
            cp.start()
        for cp in cps:
            cp.wait()

    crh, crl, cih, cil = (s[...] for s in scr_mats)
    qh, ql = _split_bf16(qth[...])
    kh, kl = _split_bf16(kth[...])
    qr = _mm3(crh, crl, qh, ql)
    qi = _mm3(cih, cil, qh, ql)
    kr = _mm3(crh, crl, kh, kl)
    ki = _mm3(cih, cil, kh, kl)
    fidx = lax.broadcasted_iota(jnp.int32, (FP, CB), 0)
    a = jnp.where((fidx == 0) | (fidx == L // 2), 1.0, 2.0).astype(f32)
    a = jnp.where(fidx <= L // 2, a, 0.0) * f32(1.0 / L)
    rr = (qr * kr + qi * ki) * a
    ri = (qi * kr - qr * ki) * a
    rrh, rrl = _split_bf16(rr)
    rih, ril = _split_bf16(ri)
    c = _mm3t(crh, crl, rrh, rrl) + _mm3t(cih, cil, rih, ril)

    iot = lax.broadcasted_iota(jnp.int32, (L, CB), 0)
    ws, ds = [], []
    for _i in range(top_k):
        mx = jnp.max(c, axis=0, keepdims=True)
        eq = c >= mx
        idx = jnp.min(jnp.where(eq, iot, L), axis=0, keepdims=True)
        ws.append(mx)
        ds.append(idx)
        c = jnp.where(iot == idx, f32(-3.0e38), c)
    w = jnp.concatenate(ws, axis=0)
    d = jnp.concatenate(ds, axis=0)
    m = jnp.max(w, axis=0, keepdims=True)
    e = jnp.exp(w - m)
    w = e / jnp.sum(e, axis=0, keepdims=True)
    pad = 16 - top_k
    w_out[...] = jnp.concatenate([w, jnp.zeros((pad, CB), f32)], axis=0)
    d_out[...] = jnp.concatenate(
        [d, jnp.zeros((pad, CB), jnp.int32)], axis=0)


@functools.lru_cache(maxsize=None)
def _dft_mats_split(L: int, FP: int):
    import ml_dtypes
    out = []
    for m in _dft_mats(L, FP)[:2]:
        hi = m.astype(ml_dtypes.bfloat16)
        lo = (m - hi.astype(np.float32)).astype(ml_dtypes.bfloat16)
        out.append(hi)
        out.append(lo)
    return tuple(out)


def _corr_topk(qth, kth, top_k, CB=256, FP=1152, interpret=False):
    L, C = qth.shape
    mats = _dft_mats_split(L, FP)
    body = functools.partial(_corr_topk_body, top_k, L, FP, CB)
    grid = (C // CB,)
    bf16 = jnp.bfloat16
    w16, d16 = pl.pallas_call(
        body,
        grid=grid,
        in_specs=[pl.BlockSpec((L, CB), lambda j: (0, j))] * 2
        + [pl.BlockSpec(memory_space=pltpu.MemorySpace.HBM)] * 4,
        out_specs=[
            pl.BlockSpec((16, CB), lambda j: (0, j)),
            pl.BlockSpec((16, CB), lambda j: (0, j)),
        ],
        out_shape=[
            jax.ShapeDtypeStruct((16, C), jnp.float32),
            jax.ShapeDtypeStruct((16, C), jnp.int32),
        ],
        scratch_shapes=[pltpu.VMEM((FP, L), bf16)] * 4
        + [pltpu.SemaphoreType.DMA],
        compiler_params=pltpu.CompilerParams(
            vmem_limit_bytes=63 * 1024 * 1024),
        interpret=interpret,
    )(qth, kth, *[jnp.asarray(m) for m in mats])
    return w16, d16


def _sc_agg(vt, wt, dt, top_k):
    C, L = vt.shape
    info = plsc.get_sparse_core_info()
    nw = info.num_cores * info.num_subcores
    cols_per = C // nw
    mesh = plsc.VectorSubcoreMesh(core_axis_name="c", subcore_axis_name="s")

    NBUF = 2

    @functools.partial(
        pl.kernel,
        out_type=jax.ShapeDtypeStruct((C, L), jnp.float32),
        mesh=mesh,
        scratch_types=[
            pltpu.VMEM((2 * L,), jnp.float32),
            pltpu.VMEM((2 * L,), jnp.float32),
            pltpu.VMEM((L,), jnp.float32),
            pltpu.VMEM((L,), jnp.float32),
            pltpu.VMEM((cols_per, 16), jnp.float32),
            pltpu.VMEM((cols_per, 16), jnp.int32),
            pltpu.SemaphoreType.DMA,
            pltpu.SemaphoreType.DMA,
            pltpu.SemaphoreType.DMA,
            pltpu.SemaphoreType.DMA,
        ],
    )
    def body(vt_hbm, wt_hbm, dt_hbm, out_hbm, vb0, vb1, ob0, ob1,
             wall, dall, si0, si1, so0, so1):
        vbufs = [vb0, vb1]
        obufs = [ob0, ob1]
        sin = [si0, si1]
        sout = [so0, so1]
        wid = lax.axis_index("s") * info.num_cores + lax.axis_index("c")
        base_col = wid * cols_per
        pltpu.sync_copy(wt_hbm.at[pl.ds(base_col, cols_per)], wall)
        pltpu.sync_copy(dt_hbm.at[pl.ds(base_col, cols_per)], dall)

        def in_copies(c0, b):
            return (
                pltpu.make_async_copy(
                    vt_hbm.at[c0], vbufs[b].at[pl.ds(0, L)], sin[b]),
                pltpu.make_async_copy(
                    vt_hbm.at[c0], vbufs[b].at[pl.ds(L, L)], sin[b]),
            )

        for b in range(NBUF):
            for cp in in_copies(base_col + b, b):
                cp.start()

        def outer(g, carry):
            for b in range(NBUF):
                j = g * NBUF + b
                c0 = base_col + j
                for cp in in_copies(c0, b):
                    cp.wait()

                @pl.when(g > 0)
                def _():
                    pltpu.make_async_copy(
                        obufs[b], out_hbm.at[c0 - NBUF], sout[b]).wait()

                wv = wall[j]
                dv = dall[j]
                wss = [wv[i] for i in range(top_k)]
                dss = [dv[i] for i in range(top_k)]
                vb = vbufs[b]
                ob = obufs[b]

                def vec_body(v, carry2):
                    base = v * 16
                    acc = wss[0] * vb[pl.ds(base + dss[0], 16)]
                    for i in range(1, top_k):
                        acc = acc + wss[i] * vb[pl.ds(base + dss[i], 16)]
                    ob[pl.ds(base, 16)] = acc
                    return carry2

                lax.fori_loop(0, L // 16, vec_body, 0, unroll=2)
                pltpu.async_copy(ob, out_hbm.at[c0], sout[b])

                @pl.when(j + NBUF < cols_per)
                def _():
                    for cp in in_copies(c0 + NBUF, b):
                        cp.start()

            return carry

        lax.fori_loop(0, cols_per // NBUF, outer, 0)
        for b in range(NBUF):
            pltpu.make_async_copy(
                obufs[b], out_hbm.at[base_col + cols_per - NBUF + b],
                sout[b]).wait()

    return body(vt, wt, dt)


def kernel(Q, K, V):
    B, H, L, D = Q.shape
    C = B * H * D
    top_k = int(_FACTOR * math.log(L))
    nh = 1
    bh = B // nh
    ch = bh * H * D

    outs = []
    for h in range(nh):
        qs = Q[h * bh:(h + 1) * bh]
        ks = K[h * bh:(h + 1) * bh]
        vs = V[h * bh:(h + 1) * bh]
        qt = jnp.transpose(qs, (2, 0, 1, 3)).reshape(L, ch)
        kt = jnp.transpose(ks, (2, 0, 1, 3)).reshape(L, ch)
        w16, d16 = _corr_topk(qt, kt, top_k)

        vt = jnp.transpose(vs, (0, 1, 3, 2)).reshape(ch, L)
        wt = jnp.transpose(w16, (1, 0))
        dt = jnp.transpose(d16, (1, 0))
        out_t = _sc_agg(vt, wt, dt, top_k)
        outs.append(
            jnp.transpose(out_t.reshape(bh, H, D, L), (0, 1, 3, 2)))
    return jnp.concatenate(outs, axis=0)
